# Initial kernel scaffold; baseline (speedup 1.0000x reference)
import jax
import jax.numpy as jnp
from jax.experimental import pallas as pl


def kernel(inputs, vocabulary):
    raise NotImplementedError("write your pallas kernel here")


# speedup vs baseline: 7.2354x; 7.2354x over previous
import functools

import jax
import jax.numpy as jnp
from jax import lax
from jax.experimental import pallas as pl
from jax.experimental.pallas import tpu as pltpu
from jax.experimental.pallas import tpu_sc as plsc

_VOCAB_SIZE = 1000
_TBL = 2048
_VOCAB_PAD = 1024
_N = 4096 * 20
_NW = 32
_PER_W = _N // _NW
_L = 16


def _lookup_body(inp_hbm, vocab_hbm, out_hbm, inp_v, vocab_v, inv_v, out_v):
    wid = lax.axis_index("s") * 2 + lax.axis_index("c")
    base = wid * _PER_W
    pltpu.sync_copy(inp_hbm.at[pl.ds(base, _PER_W)], inp_v)
    pltpu.sync_copy(vocab_hbm, vocab_v)

    oov = jnp.full((_L,), _VOCAB_SIZE, jnp.int32)

    def init_step(i, carry):
        inv_v[pl.ds(i * _L, _L)] = oov
        return carry

    lax.fori_loop(0, _TBL // _L, init_step, 0, unroll=8)

    lane = lax.iota(jnp.int32, _L)

    def scatter_step(j, carry):
        off = j * _L
        keys = vocab_v[pl.ds(off, _L)]
        plsc.store_scatter(inv_v, [keys], lane + off)
        return carry

    lax.fori_loop(0, _VOCAB_PAD // _L, scatter_step, 0, unroll=8)

    def gather_step(i, carry):
        off = i * _L
        toks = inp_v[pl.ds(off, _L)]
        out_v[pl.ds(off, _L)] = plsc.load_gather(inv_v, [toks])
        return carry

    lax.fori_loop(0, _PER_W // _L, gather_step, 0, unroll=8)

    pltpu.sync_copy(out_v, out_hbm.at[pl.ds(base, _PER_W)])


@jax.jit
def _lookup(flat_inputs, vocab_padded):
    with jax.enable_x64(False):
        mesh = plsc.VectorSubcoreMesh(core_axis_name="c", subcore_axis_name="s")
        run = pl.kernel(
            _lookup_body,
            out_type=jax.ShapeDtypeStruct((_N,), jnp.int32),
            mesh=mesh,
            scratch_types=[
                pltpu.VMEM((_PER_W,), jnp.int32),
                pltpu.VMEM((_VOCAB_PAD,), jnp.int32),
                pltpu.VMEM((_TBL,), jnp.int32),
                pltpu.VMEM((_PER_W,), jnp.int32),
            ],
            compiler_params=pltpu.CompilerParams(needs_layout_passes=False),
        )
        return run(flat_inputs, vocab_padded)


def kernel(inputs, vocabulary):
    flat = inputs.reshape(-1).astype(jnp.int32)
    vocab = vocabulary.astype(jnp.int32)
    vocab_padded = jnp.concatenate(
        [vocab, jnp.full((_VOCAB_PAD - _VOCAB_SIZE,), _TBL - 1, jnp.int32)]
    )
    out = _lookup(flat, vocab_padded)
    return out.astype(inputs.dtype).reshape(inputs.shape)
